# baseline (speedup 1.0000x reference)
import functools

import jax
import jax.numpy as jnp
from jax import lax
from jax.experimental import pallas as pl
from jax.experimental.pallas import tpu as pltpu
from jax.experimental.pallas import tpu_sc as plsc

_ENTRY = 8
_INDEX = 0
_R, _C = 16384, 128
_N = _R * _C
_NW = 32
_PER_W = _N // _NW
_BUF = 8192
_NDMA = _PER_W // _BUF


def _fill_body(out_hbm, cnt_hbm, out_buf, cnt_buf, sem):
    out_val = lax.rem(jnp.int32(_INDEX), jnp.int32(2))
    cnt_val = lax.fori_loop(
        0, _ENTRY, lambda i, s: s + lax.rem(i, 2), jnp.int32(0))
    out_vec = jnp.broadcast_to(out_val.astype(jnp.float32), (16,))
    cnt_vec = jnp.broadcast_to(cnt_val.astype(jnp.float32), (16,))

    def store(i, carry):
        out_buf[pl.ds(i * 16, 16)] = out_vec
        cnt_buf[pl.ds(i * 16, 16)] = cnt_vec
        return carry

    lax.fori_loop(0, _BUF // 16, store, 0)

    wid = lax.axis_index("s") * 2 + lax.axis_index("c")
    base = wid * _PER_W
    copies = []
    for j in range(_NDMA):
        copies.append(
            pltpu.async_copy(out_buf, out_hbm.at[pl.ds(base + j * _BUF, _BUF)], sem))
        copies.append(
            pltpu.async_copy(cnt_buf, cnt_hbm.at[pl.ds(base + j * _BUF, _BUF)], sem))
    for c in copies:
        c.wait()


_sc_fill = functools.partial(
    pl.kernel,
    out_type=(
        jax.ShapeDtypeStruct((_N,), jnp.float32),
        jax.ShapeDtypeStruct((_N,), jnp.float32),
    ),
    mesh=plsc.VectorSubcoreMesh(core_axis_name="c", subcore_axis_name="s"),
    scratch_types=[
        pltpu.VMEM((_BUF,), jnp.float32),
        pltpu.VMEM((_BUF,), jnp.float32),
        pltpu.SemaphoreType.DMA,
    ],
)(_fill_body)


def kernel(input):
    out_flat, cnt_flat = _sc_fill()
    return out_flat.reshape(_R, _C), cnt_flat.reshape(_R, _C)


# speedup vs baseline: 1.0683x; 1.0683x over previous
import functools

import jax
import jax.numpy as jnp
from jax import lax
from jax.experimental import pallas as pl
from jax.experimental.pallas import tpu as pltpu
from jax.experimental.pallas import tpu_sc as plsc

_ENTRY = 8
_INDEX = 0
_R, _C = 16384, 128
_N = _R * _C
_NW = 32
_PER_W = _N // _NW
_BUF = 4096
_NDMA = _PER_W // _BUF


def _fill_body(out_hbm, cnt_hbm, out_buf, cnt_buf, sem):
    out_val = lax.rem(jnp.int32(_INDEX), jnp.int32(2))
    cnt_val = lax.fori_loop(
        0, _ENTRY, lambda i, s: s + lax.rem(i, 2), jnp.int32(0))
    out_vec = jnp.broadcast_to(out_val.astype(jnp.float32), (16,))
    cnt_vec = jnp.broadcast_to(cnt_val.astype(jnp.float32), (16,))

    def fill(buf, vec):
        def store(i, carry):
            base = i * 64
            buf[pl.ds(base, 16)] = vec
            buf[pl.ds(base + 16, 16)] = vec
            buf[pl.ds(base + 32, 16)] = vec
            buf[pl.ds(base + 48, 16)] = vec
            return carry
        lax.fori_loop(0, _BUF // 64, store, 0)

    wid = lax.axis_index("s") * 2 + lax.axis_index("c")
    base = wid * _PER_W
    copies = []
    fill(out_buf, out_vec)
    for j in range(_NDMA):
        copies.append(
            pltpu.async_copy(out_buf, out_hbm.at[pl.ds(base + j * _BUF, _BUF)], sem))
    fill(cnt_buf, cnt_vec)
    for j in range(_NDMA):
        copies.append(
            pltpu.async_copy(cnt_buf, cnt_hbm.at[pl.ds(base + j * _BUF, _BUF)], sem))
    for c in copies:
        c.wait()


_sc_fill = functools.partial(
    pl.kernel,
    out_type=(
        jax.ShapeDtypeStruct((_N,), jnp.float32),
        jax.ShapeDtypeStruct((_N,), jnp.float32),
    ),
    mesh=plsc.VectorSubcoreMesh(core_axis_name="c", subcore_axis_name="s"),
    scratch_types=[
        pltpu.VMEM((_BUF,), jnp.float32),
        pltpu.VMEM((_BUF,), jnp.float32),
        pltpu.SemaphoreType.DMA,
    ],
)(_fill_body)


def kernel(input):
    out_flat, cnt_flat = _sc_fill()
    return out_flat.reshape(_R, _C), cnt_flat.reshape(_R, _C)


# speedup vs baseline: 1.0888x; 1.0192x over previous
import functools

import jax
import jax.numpy as jnp
from jax import lax
from jax.experimental import pallas as pl
from jax.experimental.pallas import tpu as pltpu
from jax.experimental.pallas import tpu_sc as plsc

_ENTRY = 8
_INDEX = 0
_R, _C = 16384, 128
_N = _R * _C
_NW = 32
_PER_W = _N // _NW
_BUF = 4096
_NDMA = _PER_W // _BUF


def _fill_body(out_hbm, cnt_hbm, out_buf, cnt_buf, sem):
    out_val = lax.rem(jnp.int32(_INDEX), jnp.int32(2))
    cnt_val = lax.fori_loop(
        0, _ENTRY, lambda i, s: s + lax.rem(i, 2), jnp.int32(0))
    out_vec = jnp.broadcast_to(out_val.astype(jnp.float32), (16,))
    cnt_vec = jnp.broadcast_to(cnt_val.astype(jnp.float32), (16,))

    def fill(buf, vec):
        def store(i, carry):
            base = i * 64
            buf[pl.ds(base, 16)] = vec
            buf[pl.ds(base + 16, 16)] = vec
            buf[pl.ds(base + 32, 16)] = vec
            buf[pl.ds(base + 48, 16)] = vec
            return carry
        lax.fori_loop(0, _BUF // 64, store, 0)

    wid = lax.axis_index("s") * 2 + lax.axis_index("c")
    base = wid * _PER_W
    fill(out_buf, out_vec)
    fill(cnt_buf, cnt_vec)

    def issue(j, carry):
        pltpu.async_copy(out_buf, out_hbm.at[pl.ds(base + j * _BUF, _BUF)], sem)
        pltpu.async_copy(cnt_buf, cnt_hbm.at[pl.ds(base + j * _BUF, _BUF)], sem)
        return carry

    lax.fori_loop(0, _NDMA, issue, 0)

    def drain(j, carry):
        pltpu.make_async_copy(out_buf, out_hbm.at[pl.ds(base + j * _BUF, _BUF)], sem).wait()
        pltpu.make_async_copy(cnt_buf, cnt_hbm.at[pl.ds(base + j * _BUF, _BUF)], sem).wait()
        return carry

    lax.fori_loop(0, _NDMA, drain, 0)


_sc_fill = functools.partial(
    pl.kernel,
    out_type=(
        jax.ShapeDtypeStruct((_N,), jnp.float32),
        jax.ShapeDtypeStruct((_N,), jnp.float32),
    ),
    mesh=plsc.VectorSubcoreMesh(core_axis_name="c", subcore_axis_name="s"),
    scratch_types=[
        pltpu.VMEM((_BUF,), jnp.float32),
        pltpu.VMEM((_BUF,), jnp.float32),
        pltpu.SemaphoreType.DMA,
    ],
)(_fill_body)


def kernel(input):
    out_flat, cnt_flat = _sc_fill()
    return out_flat.reshape(_R, _C), cnt_flat.reshape(_R, _C)


# speedup vs baseline: 1.1529x; 1.0588x over previous
import functools

import jax
import jax.numpy as jnp
from jax import lax
from jax.experimental import pallas as pl
from jax.experimental.pallas import tpu as pltpu
from jax.experimental.pallas import tpu_sc as plsc

_ENTRY = 8
_INDEX = 0
_R, _C = 16384, 128
_N = _R * _C
_NW = 32
_PER_W = _N // _NW
_BUF = 4096
_NDMA = _PER_W // _BUF


def _sc_cnt_body(cnt_hbm, cnt_buf, sem):
    cnt_val = lax.fori_loop(
        0, _ENTRY, lambda i, s: s + lax.rem(i, 2), jnp.int32(0))
    cnt_vec = jnp.broadcast_to(cnt_val.astype(jnp.float32), (16,))

    def store(i, carry):
        base = i * 64
        cnt_buf[pl.ds(base, 16)] = cnt_vec
        cnt_buf[pl.ds(base + 16, 16)] = cnt_vec
        cnt_buf[pl.ds(base + 32, 16)] = cnt_vec
        cnt_buf[pl.ds(base + 48, 16)] = cnt_vec
        return carry

    lax.fori_loop(0, _BUF // 64, store, 0)

    wid = lax.axis_index("s") * 2 + lax.axis_index("c")
    base = wid * _PER_W

    def issue(j, carry):
        pltpu.async_copy(cnt_buf, cnt_hbm.at[pl.ds(base + j * _BUF, _BUF)], sem)
        return carry

    lax.fori_loop(0, _NDMA, issue, 0)

    def drain(j, carry):
        pltpu.make_async_copy(cnt_buf, cnt_hbm.at[pl.ds(base + j * _BUF, _BUF)], sem).wait()
        return carry

    lax.fori_loop(0, _NDMA, drain, 0)


_sc_cnt = functools.partial(
    pl.kernel,
    out_type=jax.ShapeDtypeStruct((_N,), jnp.float32),
    mesh=plsc.VectorSubcoreMesh(core_axis_name="c", subcore_axis_name="s"),
    scratch_types=[
        pltpu.VMEM((_BUF,), jnp.float32),
        pltpu.SemaphoreType.DMA,
    ],
)(_sc_cnt_body)


def _tc_out_body(o_ref):
    ent = lax.rem(lax.broadcasted_iota(jnp.int32, o_ref.shape, 0) + _INDEX, 2)
    sel = jnp.where(lax.broadcasted_iota(jnp.int32, o_ref.shape, 0) == 0,
                    ent, jnp.zeros_like(ent))
    col = jnp.max(sel, axis=0, keepdims=True)
    o_ref[...] = jnp.broadcast_to(col, o_ref.shape).astype(jnp.float32)


_TC_BLOCKS = 8
_tc_out = pl.pallas_call(
    _tc_out_body,
    out_shape=jax.ShapeDtypeStruct((_R, _C), jnp.float32),
    grid=(_TC_BLOCKS,),
    out_specs=pl.BlockSpec((_R // _TC_BLOCKS, _C), lambda i: (i, 0)),
)


def kernel(input):
    cnt_flat = _sc_cnt()
    out = _tc_out()
    return out, cnt_flat.reshape(_R, _C)
